# baseline (speedup 1.0000x reference)
import jax
import jax.numpy as jnp
from jax.experimental import pallas as pl
from jax.experimental.pallas import tpu as pltpu

_LOG2E_HALF = 0.7213475204444817


def _cim_kernel(x_ref, w_ref, o_ref):
    x = x_ref[...]
    w = w_ref[...]
    xn = jnp.sum(x * x, axis=1, keepdims=True)
    wn = jnp.sum(w * w, axis=1)[None, :]
    dot = jax.lax.dot_general(
        x.astype(jnp.bfloat16), w.astype(jnp.bfloat16),
        (((1,), (1,)), ((), ())),
        preferred_element_type=jnp.float32,
    )
    sq = jnp.maximum(xn + wn - 2.0 * dot, 0.0)
    t = (1.0 + 1e-8) - jnp.exp2(sq * -_LOG2E_HALF)
    cim = t * jax.lax.rsqrt(t)
    o_ref[...] = cim.reshape(o_ref.shape)


def kernel(x, weights):
    b, d = x.shape
    r, c, _ = weights.shape
    n = r * c
    w2 = weights.reshape(n, d)
    nb = n // 2
    out = pl.pallas_call(
        _cim_kernel,
        grid=(2,),
        in_specs=[
            pl.BlockSpec((b, d), lambda i: (0, 0)),
            pl.BlockSpec((nb, d), lambda i: (i, 0)),
        ],
        out_specs=pl.BlockSpec((b, r // 2, c), lambda i: (0, i, 0)),
        out_shape=jax.ShapeDtypeStruct((b, r, c), jnp.float32),
        compiler_params=pltpu.CompilerParams(
            dimension_semantics=("parallel",),
        ),
    )(x, w2)
    return out


# speedup vs baseline: 2.1778x; 2.1778x over previous
import jax
import jax.numpy as jnp
from jax.experimental import pallas as pl
from jax.experimental.pallas import tpu as pltpu

_LOG2E_HALF = 0.7213475204444817


def _cim_kernel(x_ref, w_ref, o_ref):
    x = x_ref[...]
    w = w_ref[...]
    xn = jnp.sum(x * x, axis=1, keepdims=True)
    wn = jnp.sum(w * w, axis=1)[None, :]
    dot = jax.lax.dot_general(
        x.astype(jnp.bfloat16), w.astype(jnp.bfloat16),
        (((1,), (1,)), ((), ())),
        preferred_element_type=jnp.float32,
    )
    sq = jnp.maximum(xn + wn - 2.0 * dot, 0.0)
    t = (1.0 + 1e-8) - jnp.exp2(sq * -_LOG2E_HALF)
    o_ref[...] = (t * jax.lax.rsqrt(t)).astype(jnp.bfloat16)


def kernel(x, weights):
    b, d = x.shape
    r, c, _ = weights.shape
    n = r * c
    w2 = weights.reshape(n, d)
    nb = n // 2
    out = pl.pallas_call(
        _cim_kernel,
        grid=(2,),
        in_specs=[
            pl.BlockSpec((b, d), lambda i: (0, 0)),
            pl.BlockSpec((nb, d), lambda i: (i, 0)),
        ],
        out_specs=pl.BlockSpec((b, nb), lambda i: (0, i)),
        out_shape=jax.ShapeDtypeStruct((b, n), jnp.bfloat16),
        compiler_params=pltpu.CompilerParams(
            dimension_semantics=("parallel",),
        ),
    )(x, w2)
    return out.reshape(b, r, c).astype(jnp.float32)


# speedup vs baseline: 2.2012x; 1.0107x over previous
import jax
import jax.numpy as jnp
from jax.experimental import pallas as pl
from jax.experimental.pallas import tpu as pltpu

_LOG2E_HALF = 0.7213475204444817


def _cim_kernel(x_ref, w_ref, o_ref):
    x = x_ref[...]
    w = w_ref[...]
    xn = jnp.sum(x * x, axis=1, keepdims=True)
    wn = jnp.sum(w * w, axis=1)[None, :]
    dot = jax.lax.dot_general(
        x.astype(jnp.bfloat16), w.astype(jnp.bfloat16),
        (((1,), (1,)), ((), ())),
        preferred_element_type=jnp.float32,
    )
    sq = jnp.maximum(xn + wn - 2.0 * dot, 0.0)
    t = (1.0 + 1e-8) - jnp.exp2(sq * -_LOG2E_HALF)
    o_ref[...] = (t * jax.lax.rsqrt(t)).astype(jnp.bfloat16)


def kernel(x, weights):
    b, d = x.shape
    r, c, _ = weights.shape
    n = r * c
    w2 = weights.reshape(n, d)
    out = pl.pallas_call(
        _cim_kernel,
        out_shape=jax.ShapeDtypeStruct((b, n), jnp.bfloat16),
    )(x, w2)
    return out.reshape(b, r, c).astype(jnp.float32)
